# baseline (speedup 1.0000x reference)
import jax
import jax.numpy as jnp
from jax import lax
from jax.experimental import pallas as pl
from jax.experimental.pallas import tpu as pltpu
from jax.experimental.pallas import tpu_sc as plsc

N = 4096
D = 128
KNN = 8
RT = 1024
NT = N // RT
NEG = -3.0e38
NEG2 = -2.0e38
EPS = 1e-12

NUM_SUBCORES = 16
LANES = 16
ROWS_PER_TILE = N // NUM_SUBCORES


def _agg_kernel(feats_ref, wq_ref, bq_ref, wkv_ref, bkv_ref,
                w_out_ref, v_out_ref, nq_ref, nk_ref, qn_ref):
    f = feats_ref[0].astype(jnp.bfloat16)
    q = jnp.dot(f, wq_ref[...].astype(jnp.bfloat16),
                preferred_element_type=jnp.float32) + bq_ref[...]
    kv = jnp.dot(f, wkv_ref[...].astype(jnp.bfloat16),
                 preferred_element_type=jnp.float32) + bkv_ref[...]
    km = kv[:, :D]
    v_out_ref[0] = kv[:, D:]

    qn = jnp.maximum(jnp.sqrt(jnp.sum(q * q, axis=1, keepdims=True)), EPS)
    kn = jnp.maximum(jnp.sqrt(jnp.sum(km * km, axis=1, keepdims=True)), EPS)
    nq_ref[...] = (q / qn).astype(jnp.bfloat16)
    nk_ref[...] = (km / kn).astype(jnp.bfloat16)
    qn_ref[...] = qn

    kn2_row = lax.dot_general(jnp.ones((1, D), jnp.float32), km * km,
                              (((1,), (1,)), ((), ())),
                              preferred_element_type=jnp.float32,
                              precision=lax.Precision.HIGHEST)
    ka_row = jnp.maximum(jnp.sqrt(kn2_row), EPS) * (1.0 / (D ** 0.5))

    eye_rt = jnp.where(
        lax.broadcasted_iota(jnp.int32, (RT, RT), 0)
        == lax.broadcasted_iota(jnp.int32, (RT, RT), 1),
        1.0, 0.0).astype(jnp.float32)

    def tile_body(i, w):
        nqt = nq_ref[pl.ds(i * RT, RT), :]
        sim = lax.dot_general(nqt, nk_ref[...], (((1,), (1,)), ((), ())),
                              preferred_element_type=jnp.float32)
        t1 = sim * qn_ref[pl.ds(i * RT, RT), :]
        colj = lax.broadcasted_iota(jnp.int32, (RT, N), 1)
        rowi = lax.broadcasted_iota(jnp.int32, (RT, N), 0) + i * RT
        r = jnp.where(colj == rowi, NEG2, t1)

        for _ in range(KNN):
            m = jnp.max(r, axis=1, keepdims=True)
            r = jnp.where(r == m, NEG, r)

        selmask = r == NEG
        g = t1 * ka_row
        expg = jnp.where(selmask, jnp.exp(g), 0.0)
        inv_zr = 1.0 / jnp.sum(expg, axis=1, keepdims=True)
        inv_zr_t = lax.dot_general(inv_zr, eye_rt, (((0,), (0,)), ((), ())),
                                   preferred_element_type=jnp.float32)
        winc = lax.dot_general(inv_zr_t, expg, (((1,), (0,)), ((), ())),
                               preferred_element_type=jnp.float32)
        return w + winc

    w = lax.fori_loop(0, NT, tile_body, jnp.zeros((1, N), jnp.float32))
    b = pl.program_id(0)
    w_out_ref[pl.ds(b, 1), :] = w * (1.0 / N)


def _tc_stage(feats, Wq, bq, Wkv, bkv):
    B = feats.shape[0]
    return pl.pallas_call(
        _agg_kernel,
        grid=(B,),
        in_specs=[
            pl.BlockSpec((1, N, D), lambda b: (b, 0, 0)),
            pl.BlockSpec((D, D), lambda b: (0, 0)),
            pl.BlockSpec((D,), lambda b: (0,)),
            pl.BlockSpec((D, 2 * D), lambda b: (0, 0)),
            pl.BlockSpec((2 * D,), lambda b: (0,)),
        ],
        out_specs=[
            pl.BlockSpec((B, N), lambda b: (0, 0)),
            pl.BlockSpec((1, N, D), lambda b: (b, 0, 0)),
        ],
        out_shape=[
            jax.ShapeDtypeStruct((B, N), jnp.float32),
            jax.ShapeDtypeStruct((B, N, D), jnp.float32),
        ],
        scratch_shapes=[
            pltpu.VMEM((N, D), jnp.bfloat16),
            pltpu.VMEM((N, D), jnp.bfloat16),
            pltpu.VMEM((N, 1), jnp.float32),
        ],
    )(feats, Wq, bq, Wkv, bkv)


def _sc_aggregate(w_all, v_all):
    B = w_all.shape[0]
    mesh = plsc.VectorSubcoreMesh(core_axis_name="c", subcore_axis_name="s")

    @pl.kernel(
        mesh=mesh,
        out_type=jax.ShapeDtypeStruct((B, D), jnp.float32),
        scratch_types=[
            pltpu.VMEM((ROWS_PER_TILE, D), jnp.float32),
            pltpu.VMEM((ROWS_PER_TILE,), jnp.float32),
            pltpu.VMEM((1, D), jnp.float32),
            pltpu.VMEM((NUM_SUBCORES, D), jnp.float32),
            pltpu.VMEM((D,), jnp.float32),
            pltpu.VMEM_SHARED((NUM_SUBCORES, D), jnp.float32),
        ],
    )
    def sc_kernel(w_hbm, v_hbm, out_hbm, v_v, w_v, acc_v, parts_v, red_v,
                  shared):
        c = lax.axis_index("c")
        s = lax.axis_index("s")
        bag = jnp.minimum(c, B - 1)
        base = s * ROWS_PER_TILE
        pltpu.sync_copy(v_hbm.at[bag, pl.ds(base, ROWS_PER_TILE)], v_v)
        pltpu.sync_copy(w_hbm.at[bag, pl.ds(base, ROWS_PER_TILE)], w_v)

        def group_body(gi, acc):
            w16 = w_v[pl.ds(gi * LANES, LANES)]
            gdn = lax.GatherDimensionNumbers(
                offset_dims=(), collapsed_slice_dims=(0,),
                start_index_map=(0,))
            for l in range(LANES):
                wb = lax.gather(
                    w16, jnp.full((LANES, 1), l, jnp.int32), gdn, (1,),
                    mode=lax.GatherScatterMode.PROMISE_IN_BOUNDS)
                row = gi * LANES + l
                acc = tuple(
                    acc[ch] + wb * v_v[row, pl.ds(ch * LANES, LANES)]
                    for ch in range(D // LANES))
            return acc

        acc0 = tuple(jnp.zeros((LANES,), jnp.float32)
                     for _ in range(D // LANES))
        acc = lax.fori_loop(0, ROWS_PER_TILE // LANES, group_body, acc0)
        for ch in range(D // LANES):
            acc_v[0, pl.ds(ch * LANES, LANES)] = acc[ch]
        pltpu.sync_copy(acc_v, shared.at[pl.ds(s, 1)])
        plsc.subcore_barrier()

        @pl.when(s == 0)
        def _():
            pltpu.sync_copy(shared, parts_v)
            for ch in range(D // LANES):
                tot = jnp.zeros((LANES,), jnp.float32)
                for t in range(NUM_SUBCORES):
                    tot = tot + parts_v[t, pl.ds(ch * LANES, LANES)]
                red_v[pl.ds(ch * LANES, LANES)] = tot
            pltpu.sync_copy(red_v, out_hbm.at[bag])

    return sc_kernel(w_all, v_all)


@jax.jit
def kernel(feats, Wq, bq, Wkv, bkv):
    if feats.ndim == 2:
        feats = feats[None]
    w_all, v_all = _tc_stage(feats, Wq, bq, Wkv, bkv)
    return _sc_aggregate(w_all, v_all)


# speedup vs baseline: 1.0037x; 1.0037x over previous
import jax
import jax.numpy as jnp
from jax import lax
from jax.experimental import pallas as pl
from jax.experimental.pallas import tpu as pltpu
from jax.experimental.pallas import tpu_sc as plsc

N = 4096
D = 128
KNN = 8
RT = 1024
NT = N // RT
NEG = -3.0e38
NEG2 = -2.0e38
EPS = 1e-12

NUM_SUBCORES = 16
LANES = 16
ROWS_PER_TILE = N // NUM_SUBCORES


def _agg_kernel(feats_ref, wq_ref, bq_ref, wkv_ref, bkv_ref,
                w_out_ref, v_out_ref, nq_ref, nk_ref, qn_ref):
    f = feats_ref[0].astype(jnp.bfloat16)
    q = jnp.dot(f, wq_ref[...].astype(jnp.bfloat16),
                preferred_element_type=jnp.float32) + bq_ref[...]
    kv = jnp.dot(f, wkv_ref[...].astype(jnp.bfloat16),
                 preferred_element_type=jnp.float32) + bkv_ref[...]
    km = kv[:, :D]
    v_out_ref[0] = kv[:, D:]

    qn = jnp.maximum(jnp.sqrt(jnp.sum(q * q, axis=1, keepdims=True)), EPS)
    kn = jnp.maximum(jnp.sqrt(jnp.sum(km * km, axis=1, keepdims=True)), EPS)
    nq_ref[...] = (q / qn).astype(jnp.bfloat16)
    nk_ref[...] = (km / kn).astype(jnp.bfloat16)
    qn_ref[...] = qn

    kn2_row = lax.dot_general(jnp.ones((1, D), jnp.float32), km * km,
                              (((1,), (1,)), ((), ())),
                              preferred_element_type=jnp.float32,
                              precision=lax.Precision.HIGHEST)
    ka_row = jnp.maximum(jnp.sqrt(kn2_row), EPS) * (1.0 / (D ** 0.5))

    eye_rt = jnp.where(
        lax.broadcasted_iota(jnp.int32, (RT, RT), 0)
        == lax.broadcasted_iota(jnp.int32, (RT, RT), 1),
        1.0, 0.0).astype(jnp.float32)

    def tile_body(i, w):
        nqt = nq_ref[pl.ds(i * RT, RT), :]
        sim = lax.dot_general(nqt, nk_ref[...], (((1,), (1,)), ((), ())),
                              preferred_element_type=jnp.float32)
        t1 = sim * qn_ref[pl.ds(i * RT, RT), :]
        colj = lax.broadcasted_iota(jnp.int32, (RT, N), 1)
        rowi = lax.broadcasted_iota(jnp.int32, (RT, N), 0) + i * RT
        r = jnp.where(colj == rowi, NEG2, t1)
        eg = jnp.exp(t1 * ka_row)

        for _ in range(KNN):
            m = jnp.max(r, axis=1, keepdims=True)
            r = jnp.where(r == m, NEG, r)

        selmask = r == NEG
        expg = jnp.where(selmask, eg, 0.0)
        inv_zr = 1.0 / jnp.sum(expg, axis=1, keepdims=True)
        inv_zr_t = lax.dot_general(inv_zr, eye_rt, (((0,), (0,)), ((), ())),
                                   preferred_element_type=jnp.float32)
        winc = lax.dot_general(inv_zr_t, expg, (((1,), (0,)), ((), ())),
                               preferred_element_type=jnp.float32)
        return w + winc

    w = lax.fori_loop(0, NT, tile_body, jnp.zeros((1, N), jnp.float32))
    b = pl.program_id(0)
    w_out_ref[pl.ds(b, 1), :] = w * (1.0 / N)


def _tc_stage(feats, Wq, bq, Wkv, bkv):
    B = feats.shape[0]
    return pl.pallas_call(
        _agg_kernel,
        grid=(B,),
        in_specs=[
            pl.BlockSpec((1, N, D), lambda b: (b, 0, 0)),
            pl.BlockSpec((D, D), lambda b: (0, 0)),
            pl.BlockSpec((D,), lambda b: (0,)),
            pl.BlockSpec((D, 2 * D), lambda b: (0, 0)),
            pl.BlockSpec((2 * D,), lambda b: (0,)),
        ],
        out_specs=[
            pl.BlockSpec((B, N), lambda b: (0, 0)),
            pl.BlockSpec((1, N, D), lambda b: (b, 0, 0)),
        ],
        out_shape=[
            jax.ShapeDtypeStruct((B, N), jnp.float32),
            jax.ShapeDtypeStruct((B, N, D), jnp.float32),
        ],
        scratch_shapes=[
            pltpu.VMEM((N, D), jnp.bfloat16),
            pltpu.VMEM((N, D), jnp.bfloat16),
            pltpu.VMEM((N, 1), jnp.float32),
        ],
    )(feats, Wq, bq, Wkv, bkv)


def _sc_aggregate(w_all, v_all):
    B = w_all.shape[0]
    mesh = plsc.VectorSubcoreMesh(core_axis_name="c", subcore_axis_name="s")

    @pl.kernel(
        mesh=mesh,
        out_type=jax.ShapeDtypeStruct((B, D), jnp.float32),
        scratch_types=[
            pltpu.VMEM((ROWS_PER_TILE, D), jnp.float32),
            pltpu.VMEM((ROWS_PER_TILE,), jnp.float32),
            pltpu.VMEM((1, D), jnp.float32),
            pltpu.VMEM((NUM_SUBCORES, D), jnp.float32),
            pltpu.VMEM((D,), jnp.float32),
            pltpu.VMEM_SHARED((NUM_SUBCORES, D), jnp.float32),
        ],
    )
    def sc_kernel(w_hbm, v_hbm, out_hbm, v_v, w_v, acc_v, parts_v, red_v,
                  shared):
        c = lax.axis_index("c")
        s = lax.axis_index("s")
        bag = jnp.minimum(c, B - 1)
        base = s * ROWS_PER_TILE
        pltpu.sync_copy(v_hbm.at[bag, pl.ds(base, ROWS_PER_TILE)], v_v)
        pltpu.sync_copy(w_hbm.at[bag, pl.ds(base, ROWS_PER_TILE)], w_v)

        def group_body(gi, acc):
            w16 = w_v[pl.ds(gi * LANES, LANES)]
            gdn = lax.GatherDimensionNumbers(
                offset_dims=(), collapsed_slice_dims=(0,),
                start_index_map=(0,))
            for l in range(LANES):
                wb = lax.gather(
                    w16, jnp.full((LANES, 1), l, jnp.int32), gdn, (1,),
                    mode=lax.GatherScatterMode.PROMISE_IN_BOUNDS)
                row = gi * LANES + l
                acc = tuple(
                    acc[ch] + wb * v_v[row, pl.ds(ch * LANES, LANES)]
                    for ch in range(D // LANES))
            return acc

        acc0 = tuple(jnp.zeros((LANES,), jnp.float32)
                     for _ in range(D // LANES))
        acc = lax.fori_loop(0, ROWS_PER_TILE // LANES, group_body, acc0)
        for ch in range(D // LANES):
            acc_v[0, pl.ds(ch * LANES, LANES)] = acc[ch]
        pltpu.sync_copy(acc_v, shared.at[pl.ds(s, 1)])
        plsc.subcore_barrier()

        @pl.when(s == 0)
        def _():
            pltpu.sync_copy(shared, parts_v)
            for ch in range(D // LANES):
                tot = jnp.zeros((LANES,), jnp.float32)
                for t in range(NUM_SUBCORES):
                    tot = tot + parts_v[t, pl.ds(ch * LANES, LANES)]
                red_v[pl.ds(ch * LANES, LANES)] = tot
            pltpu.sync_copy(red_v, out_hbm.at[bag])

    return sc_kernel(w_all, v_all)


@jax.jit
def kernel(feats, Wq, bq, Wkv, bkv):
    if feats.ndim == 2:
        feats = feats[None]
    w_all, v_all = _tc_stage(feats, Wq, bq, Wkv, bkv)
    return _sc_aggregate(w_all, v_all)
